# initial kernel scaffold (unmeasured)
import functools

import jax
import jax.numpy as jnp
from jax import lax
from jax.experimental import pallas as pl
from jax.experimental.pallas import tpu as pltpu

Y = 4


def kernel(Q, K, V, bt, lens):
    B, _, H, D = Q.shape
    P_loc, BS = K.shape[0], K.shape[1]
    T = P_loc * BS
    NB = bt.shape[1]
    scale = D ** -0.5

    Qt = jnp.transpose(Q[:, 0, :, :], (1, 0, 2))
    Kt = jnp.transpose(K.reshape(T, H, D), (1, 0, 2))
    Vt = jnp.transpose(V.reshape(T, H, D), (1, 0, 2))
    lens2 = lens.reshape(B, 1)

    def body(q_ref, k_ref, v_ref, bt_ref, lens_ref, out_ref,
             comm_ref, send_sems, recv_sems):
        my_x = lax.axis_index("x")
        my_y = lax.axis_index("y")
        my_z = lax.axis_index("z")

        barrier_sem = pltpu.get_barrier_semaphore()
        for d in range(1, Y):
            pl.semaphore_signal(
                barrier_sem, inc=1,
                device_id=(my_x, (my_y + d) % Y, my_z),
                device_id_type=pl.DeviceIdType.MESH,
            )
        pl.semaphore_wait(barrier_sem, Y - 1)

        bt3 = bt_ref[...][:, :, None]
        tok = lax.broadcasted_iota(jnp.int32, (B, NB, T), 2)
        page = tok // BS + my_y * P_loc
        slot = lax.broadcasted_iota(jnp.int32, (B, NB, T), 1)
        lens3 = lens_ref[...].reshape(B, 1, 1)
        hit = (bt3 == page) & (slot < lens3)
        cb = jnp.sum(hit.astype(jnp.float32), axis=1)
        valid = cb > 0.0

        for h in range(H):
            qh = q_ref[h].astype(jnp.bfloat16)
            kh = k_ref[h].astype(jnp.bfloat16)
            s = lax.dot_general(
                qh, kh, (((1,), (1,)), ((), ())),
                preferred_element_type=jnp.float32,
            ) * scale
            sm = jnp.where(valid, s, -1e30)
            m = jnp.max(sm, axis=1, keepdims=True)
            p = jnp.where(valid, jnp.exp(s - m), 0.0) * cb
            l = jnp.sum(p, axis=1, keepdims=True)
            vh = v_ref[h].astype(jnp.bfloat16)
            o = lax.dot_general(
                p.astype(jnp.bfloat16), vh, (((1,), (1,)), ((), ())),
                preferred_element_type=jnp.float32,
            )
            comm_ref[0, h, :, 0:D] = o
            comm_ref[0, h, :, D:D + 1] = m
            comm_ref[0, h, :, D + 1:D + 2] = l

        sends = []
        for d in range(1, Y):
            rdma = pltpu.make_async_remote_copy(
                src_ref=comm_ref.at[0],
                dst_ref=comm_ref.at[d],
                send_sem=send_sems.at[d - 1],
                recv_sem=recv_sems.at[d],
                device_id=(my_x, (my_y + d) % Y, my_z),
                device_id_type=pl.DeviceIdType.MESH,
            )
            rdma.start()
            sends.append(rdma)

        for d in range(1, Y):
            recv = pltpu.make_async_remote_copy(
                src_ref=comm_ref.at[d],
                dst_ref=comm_ref.at[d],
                send_sem=send_sems.at[0],
                recv_sem=recv_sems.at[d],
                device_id=(my_x, (my_y + d) % Y, my_z),
                device_id_type=pl.DeviceIdType.MESH,
            )
            recv.wait_recv()

        for h in range(H):
            blocks = [comm_ref[s, h] for s in range(Y)]
            ms = [blk[:, D:D + 1] for blk in blocks]
            m_g = jnp.maximum(jnp.maximum(ms[0], ms[1]),
                              jnp.maximum(ms[2], ms[3]))
            o_acc = jnp.zeros((B, D), jnp.float32)
            l_acc = jnp.zeros((B, 1), jnp.float32)
            for s in range(Y):
                a = jnp.exp(ms[s] - m_g)
                o_acc = o_acc + a * blocks[s][:, 0:D]
                l_acc = l_acc + a * blocks[s][:, D + 1:D + 2]
            out_ref[:, 0, h, :] = o_acc / l_acc

        for rdma in sends:
            rdma.wait_send()

        @functools.partial(pl.run_scoped, exit_sem=pltpu.SemaphoreType.REGULAR)
        def _(exit_sem):
            for d in range(1, Y):
                pl.semaphore_signal(
                    exit_sem, inc=1,
                    device_id=(my_x, (my_y + d) % Y, my_z),
                    device_id_type=pl.DeviceIdType.MESH,
                )
            pl.semaphore_wait(exit_sem, Y - 1)

    return pl.pallas_call(
        body,
        out_shape=jax.ShapeDtypeStruct((B, 1, H, D), jnp.float32),
        in_specs=[pl.BlockSpec(memory_space=pltpu.VMEM)] * 5,
        out_specs=pl.BlockSpec(memory_space=pltpu.VMEM),
        scratch_shapes=[
            pltpu.VMEM((Y, H, B, 128), jnp.float32),
            pltpu.SemaphoreType.DMA((Y - 1,)),
            pltpu.SemaphoreType.DMA((Y,)),
        ],
        compiler_params=pltpu.CompilerParams(
            collective_id=0, has_side_effects=True,
        ),
    )(Qt, Kt, Vt, bt, lens2)


# baseline (device time: 19811 ns/iter reference)
import functools

import jax
import jax.numpy as jnp
from jax import lax
from jax.experimental import pallas as pl
from jax.experimental.pallas import tpu as pltpu

Y = 4


def kernel(Q, K, V, bt, lens):
    B, _, H, D = Q.shape
    P_loc, BS = K.shape[0], K.shape[1]
    T = P_loc * BS
    NB = bt.shape[1]
    scale = D ** -0.5

    Qt = jnp.transpose(Q[:, 0, :, :], (1, 0, 2))
    Kt = jnp.transpose(K.reshape(T, H, D), (1, 0, 2))
    Vt = jnp.transpose(V.reshape(T, H, D), (1, 0, 2))
    lens2 = lens.reshape(B, 1)

    def body(q_ref, k_ref, v_ref, bt_ref, lens_ref, out_ref,
             comm_ref, send_sems, recv_sems):
        my_x = lax.axis_index("x")
        my_y = lax.axis_index("y")
        my_z = lax.axis_index("z")

        barrier_sem = pltpu.get_barrier_semaphore()
        for d in range(1, Y):
            pl.semaphore_signal(
                barrier_sem, inc=1,
                device_id=(my_x, (my_y + d) % Y, my_z),
                device_id_type=pl.DeviceIdType.MESH,
            )
        pl.semaphore_wait(barrier_sem, Y - 1)

        bt3 = bt_ref[...][:, :, None]
        tok = lax.broadcasted_iota(jnp.int32, (B, NB, T), 2)
        page = tok // BS + my_y * P_loc
        slot = lax.broadcasted_iota(jnp.int32, (B, NB, T), 1)
        lens3 = lens_ref[...].reshape(B, 1, 1)
        hit = (bt3 == page) & (slot < lens3)
        cb = jnp.sum(hit.astype(jnp.float32), axis=1)
        valid = cb > 0.0

        for h in range(H):
            qh = q_ref[h].astype(jnp.bfloat16)
            kh = k_ref[h].astype(jnp.bfloat16)
            s = lax.dot_general(
                qh, kh, (((1,), (1,)), ((), ())),
                preferred_element_type=jnp.float32,
            ) * scale
            sm = jnp.where(valid, s, -1e30)
            m = jnp.max(sm, axis=1, keepdims=True)
            p = jnp.where(valid, jnp.exp(s - m), 0.0) * cb
            l = jnp.sum(p, axis=1, keepdims=True)
            vh = v_ref[h].astype(jnp.bfloat16)
            o = lax.dot_general(
                p.astype(jnp.bfloat16), vh, (((1,), (0,)), ((), ())),
                preferred_element_type=jnp.float32,
            )
            comm_ref[0, h, :, 0:D] = o
            comm_ref[0, h, :, D:D + 1] = m
            comm_ref[0, h, :, D + 1:D + 2] = l

        sends = []
        for d in range(1, Y):
            rdma = pltpu.make_async_remote_copy(
                src_ref=comm_ref.at[0],
                dst_ref=comm_ref.at[d],
                send_sem=send_sems.at[d - 1],
                recv_sem=recv_sems.at[d],
                device_id=(my_x, (my_y + d) % Y, my_z),
                device_id_type=pl.DeviceIdType.MESH,
            )
            rdma.start()
            sends.append(rdma)

        for d in range(1, Y):
            recv = pltpu.make_async_remote_copy(
                src_ref=comm_ref.at[d],
                dst_ref=comm_ref.at[d],
                send_sem=send_sems.at[0],
                recv_sem=recv_sems.at[d],
                device_id=(my_x, (my_y + d) % Y, my_z),
                device_id_type=pl.DeviceIdType.MESH,
            )
            recv.wait_recv()

        for h in range(H):
            blocks = [comm_ref[s, h] for s in range(Y)]
            ms = [blk[:, D:D + 1] for blk in blocks]
            m_g = jnp.maximum(jnp.maximum(ms[0], ms[1]),
                              jnp.maximum(ms[2], ms[3]))
            o_acc = jnp.zeros((B, D), jnp.float32)
            l_acc = jnp.zeros((B, 1), jnp.float32)
            for s in range(Y):
                a = jnp.exp(ms[s] - m_g)
                o_acc = o_acc + a * blocks[s][:, 0:D]
                l_acc = l_acc + a * blocks[s][:, D + 1:D + 2]
            out_ref[:, 0, h, :] = o_acc / l_acc

        for rdma in sends:
            rdma.wait_send()

        @functools.partial(pl.run_scoped, exit_sem=pltpu.SemaphoreType.REGULAR)
        def _(exit_sem):
            for d in range(1, Y):
                pl.semaphore_signal(
                    exit_sem, inc=1,
                    device_id=(my_x, (my_y + d) % Y, my_z),
                    device_id_type=pl.DeviceIdType.MESH,
                )
            pl.semaphore_wait(exit_sem, Y - 1)

    return pl.pallas_call(
        body,
        out_shape=jax.ShapeDtypeStruct((B, 1, H, D), jnp.float32),
        in_specs=[pl.BlockSpec(memory_space=pltpu.VMEM)] * 5,
        out_specs=pl.BlockSpec(memory_space=pltpu.VMEM),
        scratch_shapes=[
            pltpu.VMEM((Y, H, B, 128), jnp.float32),
            pltpu.SemaphoreType.DMA((Y - 1,)),
            pltpu.SemaphoreType.DMA((Y,)),
        ],
        compiler_params=pltpu.CompilerParams(
            collective_id=0, has_side_effects=True,
        ),
    )(Qt, Kt, Vt, bt, lens2)


# device time: 19204 ns/iter; 1.0316x vs baseline; 1.0316x over previous
import functools

import jax
import jax.numpy as jnp
from jax import lax
from jax.experimental import pallas as pl
from jax.experimental.pallas import tpu as pltpu

Y = 4


def kernel(Q, K, V, bt, lens):
    B, _, H, D = Q.shape
    P_loc, BS = K.shape[0], K.shape[1]
    T = P_loc * BS
    NB = bt.shape[1]
    scale = D ** -0.5

    Qt = jnp.transpose(Q[:, 0, :, :], (1, 0, 2))
    Kt = jnp.transpose(K.reshape(T, H, D), (1, 0, 2))
    Vt = jnp.transpose(V.reshape(T, H, D), (1, 0, 2))
    lens2 = lens.reshape(B, 1)

    def body(q_ref, k_ref, v_ref, bt_ref, lens_ref, out_ref,
             comm_ref, send_sems, recv_sems):
        my_x = lax.axis_index("x")
        my_y = lax.axis_index("y")
        my_z = lax.axis_index("z")

        barrier_sem = pltpu.get_barrier_semaphore()
        for d in range(1, Y):
            pl.semaphore_signal(
                barrier_sem, inc=1,
                device_id=(my_x, (my_y + d) % Y, my_z),
                device_id_type=pl.DeviceIdType.MESH,
            )
        pl.semaphore_wait(barrier_sem, Y - 1)

        bt3 = bt_ref[...][:, :, None]
        page = lax.broadcasted_iota(jnp.int32, (B, NB, P_loc), 2) \
            + my_y * P_loc
        slot = lax.broadcasted_iota(jnp.int32, (B, NB, P_loc), 1)
        lens3 = lens_ref[...].reshape(B, 1, 1)
        hit = (bt3 == page) & (slot < lens3)
        c = jnp.sum(hit.astype(jnp.float32), axis=1)
        cb = jnp.repeat(c, BS, axis=1)
        valid = cb > 0.0

        for h in range(H):
            qh = q_ref[h].astype(jnp.bfloat16)
            kh = k_ref[h].astype(jnp.bfloat16)
            s = lax.dot_general(
                qh, kh, (((1,), (1,)), ((), ())),
                preferred_element_type=jnp.float32,
            ) * scale
            sm = jnp.where(valid, s, -1e30)
            m = jnp.max(sm, axis=1, keepdims=True)
            p = jnp.where(valid, jnp.exp(s - m), 0.0) * cb
            l = jnp.sum(p, axis=1, keepdims=True)
            vh = v_ref[h].astype(jnp.bfloat16)
            o = lax.dot_general(
                p.astype(jnp.bfloat16), vh, (((1,), (0,)), ((), ())),
                preferred_element_type=jnp.float32,
            )
            comm_ref[0, h, :, 0:D] = o
            comm_ref[0, h, :, D:D + 1] = m
            comm_ref[0, h, :, D + 1:D + 2] = l

        sends = []
        for d in range(1, Y):
            rdma = pltpu.make_async_remote_copy(
                src_ref=comm_ref.at[0],
                dst_ref=comm_ref.at[d],
                send_sem=send_sems.at[d - 1],
                recv_sem=recv_sems.at[d],
                device_id=(my_x, (my_y + d) % Y, my_z),
                device_id_type=pl.DeviceIdType.MESH,
            )
            rdma.start()
            sends.append(rdma)

        for d in range(1, Y):
            recv = pltpu.make_async_remote_copy(
                src_ref=comm_ref.at[d],
                dst_ref=comm_ref.at[d],
                send_sem=send_sems.at[0],
                recv_sem=recv_sems.at[d],
                device_id=(my_x, (my_y + d) % Y, my_z),
                device_id_type=pl.DeviceIdType.MESH,
            )
            recv.wait_recv()

        for h in range(H):
            blocks = [comm_ref[s, h] for s in range(Y)]
            ms = [blk[:, D:D + 1] for blk in blocks]
            m_g = jnp.maximum(jnp.maximum(ms[0], ms[1]),
                              jnp.maximum(ms[2], ms[3]))
            o_acc = jnp.zeros((B, D), jnp.float32)
            l_acc = jnp.zeros((B, 1), jnp.float32)
            for s in range(Y):
                a = jnp.exp(ms[s] - m_g)
                o_acc = o_acc + a * blocks[s][:, 0:D]
                l_acc = l_acc + a * blocks[s][:, D + 1:D + 2]
            out_ref[:, 0, h, :] = o_acc / l_acc

        for rdma in sends:
            rdma.wait_send()

        @functools.partial(pl.run_scoped, exit_sem=pltpu.SemaphoreType.REGULAR)
        def _(exit_sem):
            for d in range(1, Y):
                pl.semaphore_signal(
                    exit_sem, inc=1,
                    device_id=(my_x, (my_y + d) % Y, my_z),
                    device_id_type=pl.DeviceIdType.MESH,
                )
            pl.semaphore_wait(exit_sem, Y - 1)

    return pl.pallas_call(
        body,
        out_shape=jax.ShapeDtypeStruct((B, 1, H, D), jnp.float32),
        in_specs=[pl.BlockSpec(memory_space=pltpu.VMEM)] * 5,
        out_specs=pl.BlockSpec(memory_space=pltpu.VMEM),
        scratch_shapes=[
            pltpu.VMEM((Y, H, B, 128), jnp.float32),
            pltpu.SemaphoreType.DMA((Y - 1,)),
            pltpu.SemaphoreType.DMA((Y,)),
        ],
        compiler_params=pltpu.CompilerParams(
            collective_id=0, has_side_effects=True,
        ),
    )(Qt, Kt, Vt, bt, lens2)


# device time: 15278 ns/iter; 1.2967x vs baseline; 1.2570x over previous
import jax
import jax.numpy as jnp
from jax import lax
from jax.experimental import pallas as pl
from jax.experimental.pallas import tpu as pltpu

Y = 4


def kernel(Q, K, V, bt, lens):
    B, _, H, D = Q.shape
    P_loc, BS = K.shape[0], K.shape[1]
    T = P_loc * BS
    NB = bt.shape[1]
    scale = D ** -0.5

    Qt = jnp.transpose(Q[:, 0, :, :], (1, 0, 2)).astype(jnp.bfloat16)
    Kt = jnp.transpose(K.reshape(T, H, D), (1, 0, 2)).astype(jnp.bfloat16)
    Vt = jnp.transpose(V.reshape(T, H, D), (1, 0, 2)).astype(jnp.bfloat16)
    lens2 = lens.reshape(B, 1)

    def body(q_ref, k_ref, v_ref, bt_ref, lens_ref, out_ref,
             comm_ref, send_sems, recv_sems):
        my_x = lax.axis_index("x")
        my_y = lax.axis_index("y")
        my_z = lax.axis_index("z")

        barrier_sem = pltpu.get_barrier_semaphore()
        for d in range(1, Y):
            pl.semaphore_signal(
                barrier_sem, inc=1,
                device_id=(my_x, (my_y + d) % Y, my_z),
                device_id_type=pl.DeviceIdType.MESH,
            )

        bt3 = bt_ref[...][:, :, None]
        page = lax.broadcasted_iota(jnp.int32, (B, NB, P_loc), 2) \
            + my_y * P_loc
        slot = lax.broadcasted_iota(jnp.int32, (B, NB, P_loc), 1)
        lens3 = lens_ref[...].reshape(B, 1, 1)
        hit = (bt3 == page) & (slot < lens3)
        c = jnp.sum(hit.astype(jnp.float32), axis=1)
        cb = jnp.repeat(c, BS, axis=1)
        valid = cb > 0.0

        for h in range(H):
            qh = q_ref[h]
            kh = k_ref[h]
            s = lax.dot_general(
                qh, kh, (((1,), (1,)), ((), ())),
                preferred_element_type=jnp.float32,
            ) * scale
            sm = jnp.where(valid, s, -1e30)
            m = jnp.max(sm, axis=1, keepdims=True)
            p = jnp.where(valid, jnp.exp(s - m), 0.0) * cb
            l = jnp.sum(p, axis=1, keepdims=True)
            vh = v_ref[h]
            o = lax.dot_general(
                p.astype(jnp.bfloat16), vh, (((1,), (0,)), ((), ())),
                preferred_element_type=jnp.float32,
            )
            comm_ref[0, h, :, 0:D] = o
            comm_ref[0, h, :, D:D + 1] = m
            comm_ref[0, h, :, D + 1:D + 2] = l

        pl.semaphore_wait(barrier_sem, Y - 1)

        sends = []
        for d in range(1, Y):
            rdma = pltpu.make_async_remote_copy(
                src_ref=comm_ref.at[0],
                dst_ref=comm_ref.at[d],
                send_sem=send_sems.at[d - 1],
                recv_sem=recv_sems.at[d],
                device_id=(my_x, (my_y + d) % Y, my_z),
                device_id_type=pl.DeviceIdType.MESH,
            )
            rdma.start()
            sends.append(rdma)

        for d in range(1, Y):
            recv = pltpu.make_async_remote_copy(
                src_ref=comm_ref.at[d],
                dst_ref=comm_ref.at[d],
                send_sem=send_sems.at[0],
                recv_sem=recv_sems.at[d],
                device_id=(my_x, (my_y + d) % Y, my_z),
                device_id_type=pl.DeviceIdType.MESH,
            )
            recv.wait_recv()

        for h in range(H):
            blocks = [comm_ref[s, h] for s in range(Y)]
            ms = [blk[:, D:D + 1] for blk in blocks]
            m_g = jnp.maximum(jnp.maximum(ms[0], ms[1]),
                              jnp.maximum(ms[2], ms[3]))
            o_acc = jnp.zeros((B, D), jnp.float32)
            l_acc = jnp.zeros((B, 1), jnp.float32)
            for s in range(Y):
                a = jnp.exp(ms[s] - m_g)
                o_acc = o_acc + a * blocks[s][:, 0:D]
                l_acc = l_acc + a * blocks[s][:, D + 1:D + 2]
            out_ref[:, 0, h, :] = o_acc / l_acc

        for rdma in sends:
            rdma.wait_send()

    return pl.pallas_call(
        body,
        out_shape=jax.ShapeDtypeStruct((B, 1, H, D), jnp.float32),
        in_specs=[pl.BlockSpec(memory_space=pltpu.VMEM)] * 5,
        out_specs=pl.BlockSpec(memory_space=pltpu.VMEM),
        scratch_shapes=[
            pltpu.VMEM((Y, H, B, 128), jnp.float32),
            pltpu.SemaphoreType.DMA((Y - 1,)),
            pltpu.SemaphoreType.DMA((Y,)),
        ],
        compiler_params=pltpu.CompilerParams(
            collective_id=0, has_side_effects=True,
        ),
    )(Qt, Kt, Vt, bt, lens2)
